# baseline (speedup 1.0000x reference)
import functools

import jax
import jax.numpy as jnp
from jax import lax
from jax.experimental import pallas as pl
from jax.experimental.pallas import tpu as pltpu
from jax.experimental.pallas import tpu_sc as plsc

DIM = 32
SPLITS = ((0, 104), (104, 96))
ROWS_PER_CHUNK = 8
T = 200
CHUNK = ROWS_PER_CHUNK * T
LANES = 16

ROW_OFFS = tuple(range(0, T - LANES + 1, LANES)) + (T - LANES,)


@functools.partial(jax.jit, static_argnums=(2,))
def _gather(idx, table, padding_idx):
    b, t = idx.shape
    assert t == T
    info = plsc.get_sparse_core_info()
    nw = info.num_cores * info.num_subcores
    rows_w = b // nw
    nch = rows_w // ROWS_PER_CHUNK
    assert rows_w % ROWS_PER_CHUNK == 0 and nch % 2 == 0

    mesh = plsc.VectorSubcoreMesh(core_axis_name="c", subcore_axis_name="s")

    @functools.partial(
        pl.kernel,
        mesh=mesh,
        compiler_params=pltpu.CompilerParams(use_tc_tiling_on_sc=False),
        out_type=jax.ShapeDtypeStruct((b, t, DIM), jnp.float32),
        scratch_types=[
            pltpu.VMEM((2, ROWS_PER_CHUNK, T), jnp.int32),
            pltpu.VMEM((2, ROWS_PER_CHUNK, T, DIM), jnp.float32),
            pltpu.SemaphoreType.DMA,
            pltpu.SemaphoreType.DMA,
            pltpu.SemaphoreType.DMA,
            pltpu.SemaphoreType.DMA,
        ],
    )
    def k(idx_hbm, table_hbm, out_hbm, idx_v, rows_v,
          gsem0, gsem1, isem0, isem1):
        gsem = (gsem0, gsem1)
        isem = (isem0, isem1)
        table = table_hbm
        c = lax.axis_index("c")
        s = lax.axis_index("s")
        wid = s * info.num_cores + c
        base = wid * rows_w

        def idx_src(g):
            r0 = pl.multiple_of(base + g * ROWS_PER_CHUNK, ROWS_PER_CHUNK)
            return idx_hbm.at[pl.ds(r0, ROWS_PER_CHUNK)]

        def gather_descs(g, buf):
            return [
                pltpu.make_async_copy(
                    table.at[idx_v.at[buf, jj, pl.ds(off, n)]],
                    rows_v.at[buf, jj, pl.ds(off, n)],
                    gsem[buf],
                )
                for jj in range(ROWS_PER_CHUNK)
                for off, n in SPLITS
            ]

        def fire_gathers(g, buf):
            for d in gather_descs(g, buf):
                d.start()

        def detect_pad(buf):
            acc = idx_v[buf, 0, pl.ds(0, LANES)] == padding_idx
            first = True
            for jj in range(ROWS_PER_CHUNK):
                for off in ROW_OFFS:
                    if first:
                        first = False
                        continue
                    vec = idx_v[buf, jj, pl.ds(off, LANES)]
                    acc = acc | (vec == padding_idx)
            acc_i = jnp.where(acc, 1, 0)
            pad = acc_i[0]
            for lane in range(1, LANES):
                pad = pad | acc_i[lane]
            return pad > 0

        def fix_pad(buf):
            zeros = jnp.zeros((LANES,), jnp.float32)

            def row_fix(jj, carry2):
                for off in ROW_OFFS:
                    vec = idx_v[buf, jj, pl.ds(off, LANES)]
                    for lane in range(LANES):
                        tok = off + lane

                        @pl.when(vec[lane] == padding_idx)
                        def _z(jj=jj, tok=tok):
                            rows_v[buf, jj, tok, pl.ds(0, LANES)] = zeros
                            rows_v[buf, jj, tok, pl.ds(LANES, LANES)] = zeros

                return carry2

            lax.fori_loop(0, ROWS_PER_CHUNK, row_fix, 0)

        def process(g, buf):
            @pl.when(g + 1 < nch)
            def _next():
                pltpu.make_async_copy(
                    idx_src(g + 1), idx_v.at[1 - buf], isem[1 - buf]
                ).wait()
                fire_gathers(g + 1, 1 - buf)

            has_pad = detect_pad(buf)
            for d in gather_descs(g, buf):
                d.wait()

            @pl.when(has_pad)
            def _fix():
                fix_pad(buf)

            @pl.when(g + 2 < nch)
            def _pref():
                pltpu.async_copy(idx_src(g + 2), idx_v.at[buf], isem[buf])

            r0 = pl.multiple_of(base + g * ROWS_PER_CHUNK, ROWS_PER_CHUNK)
            pltpu.sync_copy(rows_v.at[buf], out_hbm.at[pl.ds(r0, ROWS_PER_CHUNK)])

        pltpu.sync_copy(idx_src(0), idx_v.at[0])
        fire_gathers(0, 0)
        pltpu.async_copy(idx_src(1), idx_v.at[1], isem[1])

        def pair(p, carry):
            process(2 * p, 0)
            process(2 * p + 1, 1)
            return carry

        lax.fori_loop(0, nch // 2, pair, 0)

    return k(idx, table)


def kernel(idx, embeddings):
    mult = jnp.ones((embeddings.shape[0], 1), jnp.float32).at[0].set(0.0)
    out = _gather(idx, embeddings * mult, 0)
    return jnp.where((idx == 0)[..., None], 0.0, out)


# speedup vs baseline: 1.5968x; 1.5968x over previous
import functools

import jax
import jax.numpy as jnp
from jax import lax
from jax.experimental import pallas as pl
from jax.experimental.pallas import tpu as pltpu
from jax.experimental.pallas import tpu_sc as plsc

DIM = 32
SPLITS = ((0, 104), (104, 96))
ROWS_PER_CHUNK = 8
T = 200
CHUNK = ROWS_PER_CHUNK * T
LANES = 16

ROW_OFFS = tuple(range(0, T - LANES + 1, LANES)) + (T - LANES,)


@functools.partial(jax.jit, static_argnums=(2,))
def _gather(idx, table, padding_idx):
    b, t = idx.shape
    assert t == T
    info = plsc.get_sparse_core_info()
    nw = info.num_cores * info.num_subcores
    rows_w = b // nw
    nch = rows_w // ROWS_PER_CHUNK
    assert rows_w % ROWS_PER_CHUNK == 0 and nch % 2 == 0

    mesh = plsc.VectorSubcoreMesh(core_axis_name="c", subcore_axis_name="s")

    @functools.partial(
        pl.kernel,
        mesh=mesh,
        compiler_params=pltpu.CompilerParams(use_tc_tiling_on_sc=False),
        out_type=jax.ShapeDtypeStruct((b, t, DIM), jnp.float32),
        scratch_types=[
            pltpu.VMEM((2, ROWS_PER_CHUNK, T), jnp.int32),
            pltpu.VMEM((2, ROWS_PER_CHUNK, T, DIM), jnp.float32),
            pltpu.SemaphoreType.DMA,
            pltpu.SemaphoreType.DMA,
            pltpu.SemaphoreType.DMA,
            pltpu.SemaphoreType.DMA,
        ],
    )
    def k(idx_hbm, table_hbm, out_hbm, idx_v, rows_v,
          gsem0, gsem1, isem0, isem1):
        gsem = (gsem0, gsem1)
        isem = (isem0, isem1)
        table = table_hbm
        c = lax.axis_index("c")
        s = lax.axis_index("s")
        wid = s * info.num_cores + c
        base = wid * rows_w

        def idx_src(g):
            r0 = pl.multiple_of(base + g * ROWS_PER_CHUNK, ROWS_PER_CHUNK)
            return idx_hbm.at[pl.ds(r0, ROWS_PER_CHUNK)]

        def gather_descs(g, buf):
            return [
                pltpu.make_async_copy(
                    table.at[idx_v.at[buf, jj, pl.ds(off, n)]],
                    rows_v.at[buf, jj, pl.ds(off, n)],
                    gsem[buf],
                )
                for jj in range(ROWS_PER_CHUNK)
                for off, n in SPLITS
            ]

        def fire_gathers(g, buf):
            for d in gather_descs(g, buf):
                d.start()

        def detect_pad(buf):
            acc = idx_v[buf, 0, pl.ds(0, LANES)] == padding_idx
            first = True
            for jj in range(ROWS_PER_CHUNK):
                for off in ROW_OFFS:
                    if first:
                        first = False
                        continue
                    vec = idx_v[buf, jj, pl.ds(off, LANES)]
                    acc = acc | (vec == padding_idx)
            acc_i = jnp.where(acc, 1, 0)
            pad = acc_i[0]
            for lane in range(1, LANES):
                pad = pad | acc_i[lane]
            return pad > 0

        def fix_pad(buf):
            zeros = jnp.zeros((LANES,), jnp.float32)

            def row_fix(jj, carry2):
                for off in ROW_OFFS:
                    vec = idx_v[buf, jj, pl.ds(off, LANES)]
                    for lane in range(LANES):
                        tok = off + lane

                        @pl.when(vec[lane] == padding_idx)
                        def _z(jj=jj, tok=tok):
                            rows_v[buf, jj, tok, pl.ds(0, LANES)] = zeros
                            rows_v[buf, jj, tok, pl.ds(LANES, LANES)] = zeros

                return carry2

            lax.fori_loop(0, ROWS_PER_CHUNK, row_fix, 0)

        def process(g, buf):
            @pl.when(g + 1 < nch)
            def _next():
                pltpu.make_async_copy(
                    idx_src(g + 1), idx_v.at[1 - buf], isem[1 - buf]
                ).wait()
                fire_gathers(g + 1, 1 - buf)

            has_pad = detect_pad(buf)
            for d in gather_descs(g, buf):
                d.wait()

            @pl.when(has_pad)
            def _fix():
                fix_pad(buf)

            @pl.when(g + 2 < nch)
            def _pref():
                pltpu.async_copy(idx_src(g + 2), idx_v.at[buf], isem[buf])

            r0 = pl.multiple_of(base + g * ROWS_PER_CHUNK, ROWS_PER_CHUNK)
            pltpu.sync_copy(rows_v.at[buf], out_hbm.at[pl.ds(r0, ROWS_PER_CHUNK)])

        pltpu.sync_copy(idx_src(0), idx_v.at[0])
        fire_gathers(0, 0)
        pltpu.async_copy(idx_src(1), idx_v.at[1], isem[1])

        def pair(p, carry):
            process(2 * p, 0)
            process(2 * p + 1, 1)
            return carry

        lax.fori_loop(0, nch // 2, pair, 0)

    return k(idx, table)


def kernel(idx, embeddings):
    return _gather(idx, embeddings, 0)
